# baseline (speedup 1.0000x reference)
import jax
import jax.numpy as jnp
from jax import lax
from jax.experimental import pallas as pl
from jax.experimental.pallas import tpu as pltpu

_VOCAB = 100000
_EMBED = 128
_HIDDEN = 128
_CTX = 200

_VT = 20000
_NT = _VOCAB // _VT


def _cbow_kernel(idx_sref, table_ref, w1_ref, b1_ref, w2_ref, b2_ref, out_ref,
                 rows_v, h_ref, m_ref, s_ref, sem):
    i = pl.program_id(0)

    @pl.when(i == 0)
    def _():
        for j in range(_CTX):
            pltpu.make_async_copy(
                table_ref.at[pl.ds(idx_sref[j], 1), :],
                rows_v.at[pl.ds(j, 1), :], sem).start()
        for j in range(_CTX):
            pltpu.make_async_copy(
                table_ref.at[pl.ds(idx_sref[j], 1), :],
                rows_v.at[pl.ds(j, 1), :], sem).wait()
        emb = jnp.sum(rows_v[...], axis=0, keepdims=True)
        h = lax.dot_general(emb, w1_ref[...], (((1,), (1,)), ((), ())),
                            preferred_element_type=jnp.float32)
        h_ref[...] = jnp.maximum(h + b1_ref[...], 0.0)
        m_ref[0] = -jnp.inf
        s_ref[0] = 0.0

    logits = lax.dot_general(h_ref[...], w2_ref[...], (((1,), (1,)), ((), ())),
                             preferred_element_type=jnp.float32) + b2_ref[0]

    m_old = m_ref[0]
    m_new = jnp.maximum(m_old, jnp.max(logits))
    s_ref[0] = s_ref[0] * jnp.exp(m_old - m_new) + jnp.sum(jnp.exp(logits - m_new))
    m_ref[0] = m_new

    @pl.when(i < _NT - 1)
    def _():
        out_ref[pl.ds(i, 1), :] = logits

    @pl.when(i == _NT - 1)
    def _():
        log_z = m_ref[0] + jnp.log(s_ref[0])
        out_ref[pl.ds(0, _NT - 1), :] = out_ref[pl.ds(0, _NT - 1), :] - log_z
        out_ref[pl.ds(_NT - 1, 1), :] = logits - log_z


def kernel(inputs, emb_table, W1, b1, W2, b2):
    idx = inputs.astype(jnp.int32)
    out2d = pl.pallas_call(
        _cbow_kernel,
        grid_spec=pltpu.PrefetchScalarGridSpec(
            num_scalar_prefetch=1,
            grid=(_NT,),
            in_specs=[
                pl.BlockSpec(memory_space=pltpu.MemorySpace.HBM),
                pl.BlockSpec((_HIDDEN, _EMBED), lambda i, s: (0, 0)),
                pl.BlockSpec((1, _HIDDEN), lambda i, s: (0, 0)),
                pl.BlockSpec((_VT, _HIDDEN), lambda i, s: (i, 0)),
                pl.BlockSpec((1, 1, _VT), lambda i, s: (i, 0, 0)),
            ],
            out_specs=pl.BlockSpec((_NT, _VT), lambda i, s: (0, 0)),
            scratch_shapes=[
                pltpu.VMEM((_CTX, _EMBED), jnp.float32),
                pltpu.VMEM((1, _HIDDEN), jnp.float32),
                pltpu.SMEM((1,), jnp.float32),
                pltpu.SMEM((1,), jnp.float32),
                pltpu.SemaphoreType.DMA,
            ],
        ),
        out_shape=jax.ShapeDtypeStruct((_NT, _VT), jnp.float32),
    )(idx, emb_table, W1, b1.reshape(1, _HIDDEN), W2, b2.reshape(_NT, 1, _VT))
    return out2d.reshape(1, _VOCAB)


# speedup vs baseline: 1.0039x; 1.0039x over previous
import jax
import jax.numpy as jnp
from jax import lax
from jax.experimental import pallas as pl
from jax.experimental.pallas import tpu as pltpu

_VOCAB = 100000
_EMBED = 128
_HIDDEN = 128
_CTX = 200

_VT = 20000
_NT = _VOCAB // _VT


def _cbow_kernel(idx_sref, table_ref, w1_ref, b1_ref, w2_ref, b2_ref, out_ref,
                 rows_v, h_ref, m_ref, s_ref, sem):
    i = pl.program_id(0)

    @pl.when(i == 0)
    def _():
        for j in range(_CTX):
            pltpu.make_async_copy(
                table_ref.at[pl.ds(idx_sref[j], 1), :],
                rows_v.at[pl.ds(j, 1), :], sem).start()
        for j in range(_CTX):
            pltpu.make_async_copy(
                table_ref.at[pl.ds(idx_sref[j], 1), :],
                rows_v.at[pl.ds(j, 1), :], sem).wait()
        emb = jnp.sum(rows_v[...], axis=0, keepdims=True)
        h = lax.dot_general(emb, w1_ref[...], (((1,), (1,)), ((), ())),
                            preferred_element_type=jnp.float32)
        h_ref[...] = jnp.maximum(h + b1_ref[...], 0.0)
        m_ref[0] = -jnp.inf
        s_ref[0] = 0.0

    logits = lax.dot_general(h_ref[...], w2_ref[...], (((1,), (1,)), ((), ())),
                             preferred_element_type=jnp.float32) + b2_ref[0]
    out_ref[pl.ds(i, 1), :] = logits

    m_old = m_ref[0]
    m_new = jnp.maximum(m_old, jnp.max(logits))
    s_ref[0] = s_ref[0] * jnp.exp(m_old - m_new) + jnp.sum(jnp.exp(logits - m_new))
    m_ref[0] = m_new

    @pl.when(i == _NT - 1)
    def _():
        out_ref[...] = out_ref[...] - (m_ref[0] + jnp.log(s_ref[0]))


def kernel(inputs, emb_table, W1, b1, W2, b2):
    idx = inputs.astype(jnp.int32)
    out2d = pl.pallas_call(
        _cbow_kernel,
        grid_spec=pltpu.PrefetchScalarGridSpec(
            num_scalar_prefetch=1,
            grid=(_NT,),
            in_specs=[
                pl.BlockSpec(memory_space=pltpu.MemorySpace.HBM),
                pl.BlockSpec((_HIDDEN, _EMBED), lambda i, s: (0, 0)),
                pl.BlockSpec((1, _HIDDEN), lambda i, s: (0, 0)),
                pl.BlockSpec((_VT, _HIDDEN), lambda i, s: (i, 0)),
                pl.BlockSpec((1, 1, _VT), lambda i, s: (i, 0, 0)),
            ],
            out_specs=pl.BlockSpec((_NT, _VT), lambda i, s: (0, 0)),
            scratch_shapes=[
                pltpu.VMEM((_CTX, _EMBED), jnp.float32),
                pltpu.VMEM((1, _HIDDEN), jnp.float32),
                pltpu.SMEM((1,), jnp.float32),
                pltpu.SMEM((1,), jnp.float32),
                pltpu.SemaphoreType.DMA,
            ],
        ),
        out_shape=jax.ShapeDtypeStruct((_NT, _VT), jnp.float32),
    )(idx, emb_table, W1, b1.reshape(1, _HIDDEN), W2, b2.reshape(_NT, 1, _VT))
    return out2d.reshape(1, _VOCAB)
